# baseline (speedup 1.0000x reference)
import functools

import jax
import jax.numpy as jnp
from jax import lax
from jax.experimental import pallas as pl
from jax.experimental.pallas import tpu as pltpu
from jax.experimental.pallas import tpu_sc as plsc

NC = 2
NS = 16
NW = NC * NS
L = 16
NCHK = 8


def _make_sc_kernel(B, NNZ, Du, Dv):
    RPW = B // NW
    SLAB = RPW * NNZ
    CHUNKS = RPW // L
    CR = RPW // NCHK
    CLEN = CR * NNZ
    GC = CR // L
    mesh = plsc.VectorSubcoreMesh(core_axis_name="c", subcore_axis_name="s")

    @functools.partial(
        pl.kernel,
        mesh=mesh,
        compiler_params=pltpu.CompilerParams(
            needs_layout_passes=False, use_tc_tiling_on_sc=False),
        out_type=[jax.ShapeDtypeStruct((B,), jnp.float32)] * 4,
        scratch_types=[
            pltpu.VMEM((SLAB,), jnp.int32),
            pltpu.VMEM((SLAB,), jnp.int32),
            [pltpu.VMEM((CLEN,), jnp.float32)] * 4,
            pltpu.VMEM((RPW,), jnp.float32),
            pltpu.SemaphoreType.DMA,
            [pltpu.SemaphoreType.DMA] * 4,
        ],
    )
    def body(u_hbm, v_hbm, wu_hbm, wv_hbm,
             p_hbm, un_hbm, q_hbm, vn_hbm,
             uidx_v, vidx_v, vals, acc_v, sem_idx, gsems):
        wid = lax.axis_index("s") * NC + lax.axis_index("c")
        obase = wid * RPW

        cp_u = pltpu.async_copy(u_hbm.at[wid], uidx_v, sem_idx)
        cp_v = pltpu.async_copy(v_hbm.at[wid], vidx_v, sem_idx)

        def gather(tab, idx_ref, g, b):
            return pltpu.async_copy(
                tab.at[idx_ref.at[pl.ds(g * CLEN, CLEN)]], vals[b], gsems[b])

        cp_u.wait()
        inflight = {0: gather(wu_hbm, uidx_v, 0, 0),
                    1: gather(wu_hbm, uidx_v, 1, 1)}
        cp_v.wait()
        inflight[2] = gather(wv_hbm, vidx_v, 0, 2)
        inflight[3] = gather(wv_hbm, vidx_v, 1, 3)

        lane_nnz = lax.iota(jnp.int32, L) * NNZ

        def count_chunk(idx_ref, c, _):
            flat = c * (L * NNZ) + lane_nnz
            acc = jnp.zeros((L,), jnp.float32)
            for j in range(NNZ):
                iv = plsc.load_gather(idx_ref, [flat + j])
                acc = acc + jnp.where(iv != 0, 1.0, 0.0).astype(jnp.float32)
            acc_v[pl.ds(c * L, L)] = acc
            return _

        lax.fori_loop(0, CHUNKS, functools.partial(count_chunk, uidx_v), 0)
        pltpu.sync_copy(acc_v, un_hbm.at[pl.ds(obase, RPW)])
        lax.fori_loop(0, CHUNKS, functools.partial(count_chunk, vidx_v), 0)
        pltpu.sync_copy(acc_v, vn_hbm.at[pl.ds(obase, RPW)])

        def sum_chunk(val_ref, g, i, _):
            flat = i * (L * NNZ) + lane_nnz
            acc = jnp.zeros((L,), jnp.float32)
            for j in range(NNZ):
                acc = acc + plsc.load_gather(val_ref, [flat + j])
            acc_v[pl.ds(g * CR + i * L, L)] = acc
            return _

        def run_table(tab, idx_ref, out_hbm, b0):
            for g in range(NCHK):
                b = b0 + (g & 1)
                inflight[b].wait()
                lax.fori_loop(0, GC,
                              functools.partial(sum_chunk, vals[b], g), 0)
                if g + 2 < NCHK:
                    inflight[b] = gather(tab, idx_ref, g + 2, b)
            pltpu.sync_copy(acc_v, out_hbm.at[pl.ds(obase, RPW)])

        run_table(wu_hbm, uidx_v, p_hbm, 0)
        run_table(wv_hbm, vidx_v, q_hbm, 2)

    return body


def kernel(U, V, W_u, W_v):
    B, NNZ = U.shape
    Du = W_u.shape[0]
    Dv = W_v.shape[0]
    u_rows = U.astype(jnp.int32).reshape(NW, (B // NW) * NNZ)
    v_rows = V.astype(jnp.int32).reshape(NW, (B // NW) * NNZ)
    wu = W_u.reshape(-1)
    wv = W_v.reshape(-1)
    p, un, q, vn = _make_sc_kernel(B, NNZ, Du, Dv)(u_rows, v_rows, wu, wv)
    return p.reshape(B, 1), un, q.reshape(B, 1), vn


# speedup vs baseline: 1.0905x; 1.0905x over previous
import functools

import jax
import jax.numpy as jnp
from jax import lax
from jax.experimental import pallas as pl
from jax.experimental.pallas import tpu as pltpu
from jax.experimental.pallas import tpu_sc as plsc

NC = 2
NS = 16
NW = NC * NS
L = 16
NCHK = 16


def _make_sc_kernel(B, NNZ, Du, Dv):
    RPW = B // NW
    SLAB = RPW * NNZ
    CHUNKS = RPW // L
    CR = RPW // NCHK
    CLEN = CR * NNZ
    GC = CR // L
    TSU = (Du // NS) & ~7
    TLU = Du - NS * TSU
    mesh = plsc.VectorSubcoreMesh(core_axis_name="c", subcore_axis_name="s")

    @functools.partial(
        pl.kernel,
        mesh=mesh,
        compiler_params=pltpu.CompilerParams(
            needs_layout_passes=False, use_tc_tiling_on_sc=False),
        out_type=[jax.ShapeDtypeStruct((B,), jnp.float32)] * 4,
        scratch_types=[
            pltpu.VMEM_SHARED((Du,), jnp.float32),
            pltpu.VMEM((SLAB,), jnp.int32),
            pltpu.VMEM((SLAB,), jnp.int32),
            [pltpu.VMEM((CLEN,), jnp.float32)] * 4,
            pltpu.VMEM((RPW,), jnp.float32),
            pltpu.VMEM((RPW,), jnp.float32),
            pltpu.SemaphoreType.DMA,
            pltpu.SemaphoreType.DMA,
            pltpu.SemaphoreType.DMA,
            pltpu.SemaphoreType.DMA,
            [pltpu.SemaphoreType.DMA] * 4,
        ],
    )
    def body(u_hbm, v_hbm, wu_hbm, wv_hbm,
             p_hbm, un_hbm, q_hbm, vn_hbm,
             tab_u, uidx_v, vidx_v, vals, acc_u, acc_v,
             sem_idx_u, sem_idx_v, sem_tab, sem_tail, gsems):
        sid = lax.axis_index("s")
        wid = sid * NC + lax.axis_index("c")
        obase = wid * RPW

        tu = pltpu.async_copy(wu_hbm.at[pl.ds(sid * TSU, TSU)],
                              tab_u.at[pl.ds(sid * TSU, TSU)], sem_tab)
        tut = pltpu.make_async_copy(wu_hbm.at[pl.ds(NS * TSU, TLU)],
                                    tab_u.at[pl.ds(NS * TSU, TLU)], sem_tail)

        @pl.when(sid == 0)
        def _():
            tut.start()

        cp_u = pltpu.async_copy(u_hbm.at[wid], uidx_v, sem_idx_u)
        cp_v = pltpu.async_copy(v_hbm.at[wid], vidx_v, sem_idx_v)

        def gather(tab, idx_ref, goff, b):
            return pltpu.make_async_copy(
                tab.at[idx_ref.at[pl.ds(goff * CLEN, CLEN)]],
                vals[b], gsems[b])

        cp_v.wait()
        gather(wv_hbm, vidx_v, 0, 2).start()
        gather(wv_hbm, vidx_v, 1, 3).start()

        lane_nnz = lax.iota(jnp.int32, L) * NNZ

        def count_chunk(idx_ref, c, _):
            flat = c * (L * NNZ) + lane_nnz
            acc = jnp.zeros((L,), jnp.float32)
            for j in range(NNZ):
                iv = plsc.load_gather(idx_ref, [flat + j])
                acc = acc + jnp.where(iv != 0, 1.0, 0.0).astype(jnp.float32)
            acc_u[pl.ds(c * L, L)] = acc
            return _

        cp_u.wait()
        lax.fori_loop(0, CHUNKS, functools.partial(count_chunk, uidx_v), 0)
        pltpu.sync_copy(acc_u, un_hbm.at[pl.ds(obase, RPW)])
        lax.fori_loop(0, CHUNKS, functools.partial(count_chunk, vidx_v), 0)
        pltpu.sync_copy(acc_u, vn_hbm.at[pl.ds(obase, RPW)])

        tu.wait()

        @pl.when(sid == 0)
        def _():
            tut.wait()

        plsc.subcore_barrier()

        gather(tab_u, uidx_v, 0, 0).start()
        gather(tab_u, uidx_v, 1, 1).start()

        def sum_chunk(val_ref, acc_ref, g, i, _):
            flat = i * (L * NNZ) + lane_nnz
            acc = jnp.zeros((L,), jnp.float32)
            for j in range(NNZ):
                acc = acc + plsc.load_gather(val_ref, [flat + j])
            acc_ref[pl.ds(g * CR + i * L, L)] = acc
            return _

        def step(tab, idx_ref, acc_ref, g, b):
            gather(tab, idx_ref, g, b).wait()
            lax.fori_loop(
                0, GC, functools.partial(sum_chunk, vals[b], acc_ref, g), 0)

            @pl.when(g + 2 < NCHK)
            def _():
                gather(tab, idx_ref, g + 2, b).start()

        def pair(i, _):
            for par in range(2):
                g = 2 * i + par
                step(tab_u, uidx_v, acc_u, g, par)
                step(wv_hbm, vidx_v, acc_v, g, 2 + par)
            return _

        lax.fori_loop(0, NCHK // 2, pair, 0)
        pltpu.sync_copy(acc_u, p_hbm.at[pl.ds(obase, RPW)])
        pltpu.sync_copy(acc_v, q_hbm.at[pl.ds(obase, RPW)])

    return body


def kernel(U, V, W_u, W_v):
    B, NNZ = U.shape
    Du = W_u.shape[0]
    Dv = W_v.shape[0]
    u_rows = U.astype(jnp.int32).reshape(NW, (B // NW) * NNZ)
    v_rows = V.astype(jnp.int32).reshape(NW, (B // NW) * NNZ)
    wu = W_u.reshape(-1)
    wv = W_v.reshape(-1)
    p, un, q, vn = _make_sc_kernel(B, NNZ, Du, Dv)(u_rows, v_rows, wu, wv)
    return p.reshape(B, 1), un, q.reshape(B, 1), vn


# speedup vs baseline: 1.1025x; 1.0109x over previous
import functools

import jax
import jax.numpy as jnp
from jax import lax
from jax.experimental import pallas as pl
from jax.experimental.pallas import tpu as pltpu
from jax.experimental.pallas import tpu_sc as plsc

NC = 2
NS = 16
NW = NC * NS
L = 16
NCHK = 16


def _make_sc_kernel(B, NNZ, Du, Dv):
    RPW = B // NW
    SLAB = RPW * NNZ
    CHUNKS = RPW // L
    CR = RPW // NCHK
    CLEN = CR * NNZ
    GC = CR // L
    TSU = (Du // NS) & ~7
    TLU = Du - NS * TSU
    mesh = plsc.VectorSubcoreMesh(core_axis_name="c", subcore_axis_name="s")

    @functools.partial(
        pl.kernel,
        mesh=mesh,
        compiler_params=pltpu.CompilerParams(
            needs_layout_passes=False, use_tc_tiling_on_sc=False),
        out_type=[jax.ShapeDtypeStruct((B,), jnp.float32)] * 4,
        scratch_types=[
            pltpu.VMEM_SHARED((Du,), jnp.float32),
            pltpu.VMEM((SLAB,), jnp.int32),
            pltpu.VMEM((SLAB,), jnp.int32),
            [pltpu.VMEM((CLEN,), jnp.float32)] * 8,
            pltpu.VMEM((RPW,), jnp.float32),
            pltpu.VMEM((RPW,), jnp.float32),
            pltpu.SemaphoreType.DMA,
            pltpu.SemaphoreType.DMA,
            pltpu.SemaphoreType.DMA,
            pltpu.SemaphoreType.DMA,
            [pltpu.SemaphoreType.DMA] * 8,
        ],
    )
    def body(u_hbm, v_hbm, wu_hbm, wv_hbm,
             p_hbm, un_hbm, q_hbm, vn_hbm,
             tab_u, uidx_v, vidx_v, vals, acc_u, acc_v,
             sem_idx_u, sem_idx_v, sem_tab, sem_tail, gsems):
        sid = lax.axis_index("s")
        wid = sid * NC + lax.axis_index("c")
        obase = wid * RPW

        tu = pltpu.async_copy(wu_hbm.at[pl.ds(sid * TSU, TSU)],
                              tab_u.at[pl.ds(sid * TSU, TSU)], sem_tab)
        tut = pltpu.make_async_copy(wu_hbm.at[pl.ds(NS * TSU, TLU)],
                                    tab_u.at[pl.ds(NS * TSU, TLU)], sem_tail)

        @pl.when(sid == 0)
        def _():
            tut.start()

        cp_u = pltpu.async_copy(u_hbm.at[wid], uidx_v, sem_idx_u)
        cp_v = pltpu.async_copy(v_hbm.at[wid], vidx_v, sem_idx_v)

        def gather(tab, idx_ref, goff, b):
            return pltpu.make_async_copy(
                tab.at[idx_ref.at[pl.ds(goff * CLEN, CLEN)]],
                vals[b], gsems[b])

        cp_v.wait()
        for g in range(4):
            gather(wv_hbm, vidx_v, g, 4 + g).start()

        lane_nnz = lax.iota(jnp.int32, L) * NNZ

        def count_chunk(idx_ref, c, _):
            flat = c * (L * NNZ) + lane_nnz
            acc = jnp.zeros((L,), jnp.float32)
            for j in range(NNZ):
                iv = plsc.load_gather(idx_ref, [flat + j])
                acc = acc + jnp.where(iv != 0, 1.0, 0.0).astype(jnp.float32)
            acc_u[pl.ds(c * L, L)] = acc
            return _

        cp_u.wait()
        lax.fori_loop(0, CHUNKS, functools.partial(count_chunk, uidx_v), 0)
        pltpu.sync_copy(acc_u, un_hbm.at[pl.ds(obase, RPW)])
        lax.fori_loop(0, CHUNKS, functools.partial(count_chunk, vidx_v), 0)
        pltpu.sync_copy(acc_u, vn_hbm.at[pl.ds(obase, RPW)])

        tu.wait()

        @pl.when(sid == 0)
        def _():
            tut.wait()

        plsc.subcore_barrier()

        for g in range(4):
            gather(tab_u, uidx_v, g, g).start()

        def sum_chunk(val_ref, acc_ref, g, i, _):
            flat = i * (L * NNZ) + lane_nnz
            acc = jnp.zeros((L,), jnp.float32)
            for j in range(NNZ):
                acc = acc + plsc.load_gather(val_ref, [flat + j])
            acc_ref[pl.ds(g * CR + i * L, L)] = acc
            return _

        def step(tab, idx_ref, acc_ref, g, b):
            gather(tab, idx_ref, g, b).wait()
            lax.fori_loop(
                0, GC, functools.partial(sum_chunk, vals[b], acc_ref, g), 0)

            @pl.when(g + 4 < NCHK)
            def _():
                gather(tab, idx_ref, g + 4, b).start()

        def quad(i, _):
            for par in range(4):
                g = 4 * i + par
                step(tab_u, uidx_v, acc_u, g, par)
                step(wv_hbm, vidx_v, acc_v, g, 4 + par)
            return _

        lax.fori_loop(0, NCHK // 4, quad, 0)
        pltpu.sync_copy(acc_u, p_hbm.at[pl.ds(obase, RPW)])
        pltpu.sync_copy(acc_v, q_hbm.at[pl.ds(obase, RPW)])

    return body


def kernel(U, V, W_u, W_v):
    B, NNZ = U.shape
    Du = W_u.shape[0]
    Dv = W_v.shape[0]
    u_rows = U.astype(jnp.int32).reshape(NW, (B // NW) * NNZ)
    v_rows = V.astype(jnp.int32).reshape(NW, (B // NW) * NNZ)
    wu = W_u.reshape(-1)
    wv = W_v.reshape(-1)
    p, un, q, vn = _make_sc_kernel(B, NNZ, Du, Dv)(u_rows, v_rows, wu, wv)
    return p.reshape(B, 1), un, q.reshape(B, 1), vn


# speedup vs baseline: 1.1408x; 1.0347x over previous
import functools

import jax
import jax.numpy as jnp
from jax import lax
from jax.experimental import pallas as pl
from jax.experimental.pallas import tpu as pltpu
from jax.experimental.pallas import tpu_sc as plsc

NC = 2
NS = 16
L = 16
NCHK = 32


def _make_sc_kernel(B, NNZ, Du, Dv):
    RPW = B // NS
    SLAB = RPW * NNZ
    CHUNKS = RPW // L
    CR = RPW // NCHK
    CLEN = CR * NNZ
    GC = CR // L
    D = max(Du, Dv)
    TSU = (Du // NS) & ~7
    TLU = Du - NS * TSU
    TSV = (Dv // NS) & ~7
    TLV = Dv - NS * TSV
    mesh = plsc.VectorSubcoreMesh(core_axis_name="c", subcore_axis_name="s")

    @functools.partial(
        pl.kernel,
        mesh=mesh,
        compiler_params=pltpu.CompilerParams(
            needs_layout_passes=False, use_tc_tiling_on_sc=False),
        out_type=[jax.ShapeDtypeStruct((B,), jnp.float32)] * 4,
        scratch_types=[
            pltpu.VMEM_SHARED((D,), jnp.float32),
            pltpu.VMEM((SLAB,), jnp.int32),
            [pltpu.VMEM((CLEN,), jnp.float32)] * 8,
            pltpu.VMEM((RPW,), jnp.float32),
            pltpu.SemaphoreType.DMA,
            pltpu.SemaphoreType.DMA,
            pltpu.SemaphoreType.DMA,
            [pltpu.SemaphoreType.DMA] * 8,
        ],
    )
    def body(u_hbm, v_hbm, wu_hbm, wv_hbm,
             p_hbm, un_hbm, q_hbm, vn_hbm,
             tab, idx_v, vals, acc,
             sem_idx, sem_tab, sem_tail, gsems):
        cid = lax.axis_index("c")
        sid = lax.axis_index("s")
        obase = sid * RPW
        lane_nnz = lax.iota(jnp.int32, L) * NNZ

        def gather(idx_ref, goff, b):
            return pltpu.make_async_copy(
                tab.at[idx_ref.at[pl.ds(goff * CLEN, CLEN)]],
                vals[b], gsems[b])

        def count_chunk(idx_ref, c, _):
            flat = c * (L * NNZ) + lane_nnz
            cacc = jnp.zeros((L,), jnp.float32)
            for j in range(NNZ):
                iv = plsc.load_gather(idx_ref, [flat + j])
                cacc = cacc + jnp.where(iv != 0, 1.0, 0.0).astype(jnp.float32)
            acc[pl.ds(c * L, L)] = cacc
            return _

        def sum_chunk(val_ref, g, i, _):
            flat = i * (L * NNZ) + lane_nnz
            sacc = jnp.zeros((L,), jnp.float32)
            for j in range(NNZ):
                sacc = sacc + plsc.load_gather(val_ref, [flat + j])
            acc[pl.ds(g * CR + i * L, L)] = sacc
            return _

        def run(idx_hbm, w_hbm, TS, TL, cnt_hbm, sum_hbm):
            ts = pltpu.async_copy(w_hbm.at[pl.ds(sid * TS, TS)],
                                  tab.at[pl.ds(sid * TS, TS)], sem_tab)
            tl = pltpu.make_async_copy(w_hbm.at[pl.ds(NS * TS, TL)],
                                       tab.at[pl.ds(NS * TS, TL)], sem_tail)

            @pl.when(sid == 0)
            def _():
                tl.start()

            cp = pltpu.async_copy(idx_hbm.at[sid], idx_v, sem_idx)
            cp.wait()

            lax.fori_loop(0, CHUNKS, functools.partial(count_chunk, idx_v), 0)
            pltpu.sync_copy(acc, cnt_hbm.at[pl.ds(obase, RPW)])

            ts.wait()

            @pl.when(sid == 0)
            def _():
                tl.wait()

            plsc.subcore_barrier()

            for b in range(8):
                gather(idx_v, b, b).start()

            def step(g, b):
                gather(idx_v, g, b).wait()
                lax.fori_loop(
                    0, GC, functools.partial(sum_chunk, vals[b], g), 0)

                @pl.when(g + 8 < NCHK)
                def _():
                    gather(idx_v, g + 8, b).start()

            def octet(i, _):
                for par in range(8):
                    step(8 * i + par, par)
                return _

            lax.fori_loop(0, NCHK // 8, octet, 0)
            pltpu.sync_copy(acc, sum_hbm.at[pl.ds(obase, RPW)])

        @pl.when(cid == 0)
        def _():
            run(u_hbm, wu_hbm, TSU, TLU, un_hbm, p_hbm)

        @pl.when(cid == 1)
        def _():
            run(v_hbm, wv_hbm, TSV, TLV, vn_hbm, q_hbm)

    return body


def kernel(U, V, W_u, W_v):
    B, NNZ = U.shape
    Du = W_u.shape[0]
    Dv = W_v.shape[0]
    u_rows = U.astype(jnp.int32).reshape(NS, (B // NS) * NNZ)
    v_rows = V.astype(jnp.int32).reshape(NS, (B // NS) * NNZ)
    wu = W_u.reshape(-1)
    wv = W_v.reshape(-1)
    p, un, q, vn = _make_sc_kernel(B, NNZ, Du, Dv)(u_rows, v_rows, wu, wv)
    return p.reshape(B, 1), un, q.reshape(B, 1), vn
